# baseline (speedup 1.0000x reference)
import functools

import jax
import jax.numpy as jnp
from jax import lax
from jax.experimental import pallas as pl
from jax.experimental.pallas import tpu as pltpu
from jax.experimental.pallas import tpu_sc as plsc

N = 10000
NP = 10240
D = 128
E = 320000
NC, NS, L = 2, 16, 16
NW = NC * NS
CH = 128
CPT0 = 96
CPT1 = 64
CPT = (CPT0 + CPT1) // 2
EP = NS * (CPT0 + CPT1) * CH
RPT = NP // NS

_mesh = plsc.VectorSubcoreMesh(core_axis_name="c", subcore_axis_name="s")


def _deg_body(dst_hbm, deg_hbm, dacc, didx):
    c = lax.axis_index("c")
    s = lax.axis_index("s")
    w = s * NC + c
    z16 = jnp.zeros((L,), jnp.float32)
    ones16 = jnp.ones((L,), jnp.float32)

    @pl.loop(0, NP // L)
    def _zero(i):
        dacc[pl.ds(i * L, L)] = z16

    base = w * CPT

    @pl.loop(0, CPT)
    def _count(i):
        pltpu.sync_copy(dst_hbm.at[pl.ds((base + i) * CH, CH)], didx)
        for j in range(CH // L):
            idx = didx[pl.ds(j * L, L)]
            plsc.addupdate_scatter(dacc, [idx], ones16)

    pltpu.sync_copy(dacc, deg_hbm.at[pl.ds(w * NP, NP)])


_deg_kernel = functools.partial(
    pl.kernel,
    out_type=jax.ShapeDtypeStruct((NW * NP,), jnp.float32),
    mesh=_mesh,
    scratch_types=[
        pltpu.VMEM((NP,), jnp.float32),
        pltpu.VMEM((CH,), jnp.int32),
    ],
    compiler_params=pltpu.CompilerParams(needs_layout_passes=False),
)(_deg_body)


def _agg_body(hp_hbm, src_hbm, dst_hbm, out_hbm, sidx, didx, rows, acc, sem):
    c = lax.axis_index("c")
    s = lax.axis_index("s")
    z16 = jnp.zeros((L,), jnp.float32)

    @pl.loop(0, CH)
    def _zrow(r):
        for j in range(D // L):
            rows[r, pl.ds(j * L, L)] = z16

    @pl.loop(0, RPT // CH)
    def _zacc(j):
        pltpu.sync_copy(rows, acc.at[pl.ds(s * RPT + j * CH, CH)])

    plsc.subcore_barrier()

    def edge_loop(cpt_c, base0):
        @pl.loop(0, cpt_c)
        def _edges(i):
            eb = (base0 + i) * CH
            pltpu.sync_copy(src_hbm.at[pl.ds(eb, CH)], sidx)
            pltpu.async_copy(hp_hbm.at[sidx], rows, sem).wait()
            pltpu.sync_copy(dst_hbm.at[pl.ds(eb, CH)], didx)
            pltpu.sync_copy(rows, acc.at[didx], add=True)

    @pl.when(c == 0)
    def _c0():
        edge_loop(CPT0, s * CPT0)

    @pl.when(c == 1)
    def _c1():
        edge_loop(CPT1, NS * CPT0 + s * CPT1)

    plsc.subcore_barrier()

    @pl.loop(0, RPT // CH)
    def _wb(j):
        r0 = s * RPT + j * CH
        pltpu.sync_copy(acc.at[pl.ds(r0, CH)], rows)
        pltpu.sync_copy(rows, out_hbm.at[pl.ds(c * NP + r0, CH)])


_agg_kernel = functools.partial(
    pl.kernel,
    out_type=jax.ShapeDtypeStruct((NC * NP, D), jnp.float32),
    mesh=_mesh,
    scratch_types=[
        pltpu.VMEM((CH,), jnp.int32),
        pltpu.VMEM((CH,), jnp.int32),
        pltpu.VMEM((CH, D), jnp.float32),
        pltpu.VMEM_SHARED((NP, D), jnp.float32),
        pltpu.SemaphoreType.DMA,
    ],
)(_agg_body)


BR = 512
GRID = NP // BR


def _scale1_body(x_ref, w1_ref, degp_ref, hp1_ref, dinv_ref):
    deg = jnp.sum(degp_ref[...], axis=0) + 1.0
    di = lax.rsqrt(deg)[:, None]
    h = jnp.dot(x_ref[...], w1_ref[...], preferred_element_type=jnp.float32)
    hp1_ref[...] = h * di
    dinv_ref[...] = di


def _tc_scale1(x_p, W1, degp):
    return pl.pallas_call(
        _scale1_body,
        grid=(GRID,),
        in_specs=[
            pl.BlockSpec((BR, D), lambda i: (i, 0)),
            pl.BlockSpec((D, D), lambda i: (0, 0)),
            pl.BlockSpec((NW, BR), lambda i: (0, i)),
        ],
        out_specs=[
            pl.BlockSpec((BR, D), lambda i: (i, 0)),
            pl.BlockSpec((BR, 1), lambda i: (i, 0)),
        ],
        out_shape=[
            jax.ShapeDtypeStruct((NP, D), jnp.float32),
            jax.ShapeDtypeStruct((NP, 1), jnp.float32),
        ],
    )(x_p, W1, degp)


def _mid_body(acc_ref, hp_ref, dinv_ref, b_ref, w2_ref, hp2_ref):
    di = dinv_ref[...]
    a = acc_ref[0] + acc_ref[1] + hp_ref[...]
    o = jnp.maximum(a * di + b_ref[...], 0.0)
    hp2_ref[...] = jnp.dot(o, w2_ref[...], preferred_element_type=jnp.float32) * di


def _tc_mid(accp, hp, dinv, b, W2):
    return pl.pallas_call(
        _mid_body,
        grid=(GRID,),
        in_specs=[
            pl.BlockSpec((2, BR, D), lambda i: (0, i, 0)),
            pl.BlockSpec((BR, D), lambda i: (i, 0)),
            pl.BlockSpec((BR, 1), lambda i: (i, 0)),
            pl.BlockSpec((1, D), lambda i: (0, 0)),
            pl.BlockSpec((D, D), lambda i: (0, 0)),
        ],
        out_specs=pl.BlockSpec((BR, D), lambda i: (i, 0)),
        out_shape=jax.ShapeDtypeStruct((NP, D), jnp.float32),
    )(accp, hp, dinv, b, W2)


def _head_body(acc_ref, hp_ref, dinv_ref, b_ref, wo_ref, bo_ref, y_ref):
    a = acc_ref[0] + acc_ref[1] + hp_ref[...]
    o = jnp.maximum(a * dinv_ref[...] + b_ref[...], 0.0)
    y_ref[...] = jnp.dot(o, wo_ref[...], preferred_element_type=jnp.float32) + bo_ref[...]


def _tc_head(accp, hp, dinv, b, Wo, bo):
    return pl.pallas_call(
        _head_body,
        grid=(GRID,),
        in_specs=[
            pl.BlockSpec((2, BR, D), lambda i: (0, i, 0)),
            pl.BlockSpec((BR, D), lambda i: (i, 0)),
            pl.BlockSpec((BR, 1), lambda i: (i, 0)),
            pl.BlockSpec((1, D), lambda i: (0, 0)),
            pl.BlockSpec((D, 1), lambda i: (0, 0)),
            pl.BlockSpec((1, 1), lambda i: (0, 0)),
        ],
        out_specs=pl.BlockSpec((BR, 1), lambda i: (i, 0)),
        out_shape=jax.ShapeDtypeStruct((NP, 1), jnp.float32),
    )(accp, hp, dinv, b, Wo, bo)


def kernel(x, edge_index, W1, b1, W2, b2, Wo, bo):
    x_p = jnp.pad(x, ((0, NP - N), (0, 0)))
    pad = jnp.full((EP - E,), NP - 1, jnp.int32)
    src_p = jnp.concatenate([edge_index[0], pad])
    dst_p = jnp.concatenate([edge_index[1], pad])

    degp = _deg_kernel(dst_p).reshape(NW, NP)
    hp1, dinv = _tc_scale1(x_p, W1, degp)
    acc1 = _agg_kernel(hp1, src_p, dst_p).reshape(NC, NP, D)
    hp2 = _tc_mid(acc1, hp1, dinv, b1.reshape(1, D), W2)
    acc2 = _agg_kernel(hp2, src_p, dst_p).reshape(NC, NP, D)
    y = _tc_head(acc2, hp2, dinv, b2.reshape(1, D), Wo, bo.reshape(1, 1))
    return y[:N, 0]


# speedup vs baseline: 1.3389x; 1.3389x over previous
import functools

import jax
import jax.numpy as jnp
from jax import lax
from jax.experimental import pallas as pl
from jax.experimental.pallas import tpu as pltpu
from jax.experimental.pallas import tpu_sc as plsc

N = 10000
NP = 10240
D = 128
E = 320000
NC, NS, L = 2, 16, 16
NW = NC * NS
CH = 128
CPT = -(-E // (NW * CH))
EP = NW * CPT * CH
RPT = NP // NS

_mesh = plsc.VectorSubcoreMesh(core_axis_name="c", subcore_axis_name="s")


def _deg_body(dst_hbm, deg_hbm, dacc, didx):
    c = lax.axis_index("c")
    s = lax.axis_index("s")
    w = s * NC + c
    z16 = jnp.zeros((L,), jnp.float32)
    ones16 = jnp.ones((L,), jnp.float32)

    @pl.loop(0, NP // L)
    def _zero(i):
        dacc[pl.ds(i * L, L)] = z16

    base = w * CPT

    @pl.loop(0, CPT)
    def _count(i):
        pltpu.sync_copy(dst_hbm.at[pl.ds((base + i) * CH, CH)], didx)
        for j in range(CH // L):
            idx = didx[pl.ds(j * L, L)]
            plsc.addupdate_scatter(dacc, [idx], ones16)

    pltpu.sync_copy(dacc, deg_hbm.at[pl.ds(w * NP, NP)])


_deg_kernel = functools.partial(
    pl.kernel,
    out_type=jax.ShapeDtypeStruct((NW * NP,), jnp.float32),
    mesh=_mesh,
    scratch_types=[
        pltpu.VMEM((NP,), jnp.float32),
        pltpu.VMEM((CH,), jnp.int32),
    ],
    compiler_params=pltpu.CompilerParams(needs_layout_passes=False),
)(_deg_body)


def _agg_body(hp_hbm, src_hbm, dst_hbm, out_hbm, sidx, didx, rows, acc, sem):
    c = lax.axis_index("c")
    s = lax.axis_index("s")
    z16 = jnp.zeros((L,), jnp.float32)

    @pl.loop(0, CH)
    def _zrow(r):
        for j in range(D // L):
            rows[r, pl.ds(j * L, L)] = z16

    @pl.loop(0, RPT // CH)
    def _zacc(j):
        pltpu.sync_copy(rows, acc.at[pl.ds(s * RPT + j * CH, CH)])

    plsc.subcore_barrier()

    base = (s * NC + c) * CPT

    @pl.loop(0, CPT)
    def _edges(i):
        eb = (base + i) * CH
        pltpu.sync_copy(src_hbm.at[pl.ds(eb, CH)], sidx)
        pltpu.async_copy(hp_hbm.at[sidx], rows, sem).wait()
        pltpu.sync_copy(dst_hbm.at[pl.ds(eb, CH)], didx)
        pltpu.sync_copy(rows, acc.at[didx], add=True)

    plsc.subcore_barrier()

    @pl.loop(0, RPT // CH)
    def _wb(j):
        r0 = s * RPT + j * CH
        pltpu.sync_copy(acc.at[pl.ds(r0, CH)], rows)
        pltpu.sync_copy(rows, out_hbm.at[pl.ds(c * NP + r0, CH)])


_agg_kernel = functools.partial(
    pl.kernel,
    out_type=jax.ShapeDtypeStruct((NC * NP, D), jnp.float32),
    mesh=_mesh,
    scratch_types=[
        pltpu.VMEM((CH,), jnp.int32),
        pltpu.VMEM((CH,), jnp.int32),
        pltpu.VMEM((CH, D), jnp.float32),
        pltpu.VMEM_SHARED((NP, D), jnp.float32),
        pltpu.SemaphoreType.DMA,
    ],
)(_agg_body)


BR = 512
GRID = NP // BR


def _scale1_body(x_ref, w1_ref, degp_ref, hp1_ref, dinv_ref):
    deg = jnp.sum(degp_ref[...], axis=0) + 1.0
    di = lax.rsqrt(deg)[:, None]
    h = jnp.dot(x_ref[...], w1_ref[...], preferred_element_type=jnp.float32)
    hp1_ref[...] = h * di
    dinv_ref[...] = di


def _tc_scale1(x_p, W1, degp):
    return pl.pallas_call(
        _scale1_body,
        grid=(GRID,),
        in_specs=[
            pl.BlockSpec((BR, D), lambda i: (i, 0)),
            pl.BlockSpec((D, D), lambda i: (0, 0)),
            pl.BlockSpec((NW, BR), lambda i: (0, i)),
        ],
        out_specs=[
            pl.BlockSpec((BR, D), lambda i: (i, 0)),
            pl.BlockSpec((BR, 1), lambda i: (i, 0)),
        ],
        out_shape=[
            jax.ShapeDtypeStruct((NP, D), jnp.float32),
            jax.ShapeDtypeStruct((NP, 1), jnp.float32),
        ],
    )(x_p, W1, degp)


def _mid_body(acc_ref, hp_ref, dinv_ref, b_ref, w2_ref, hp2_ref):
    di = dinv_ref[...]
    a = acc_ref[0] + acc_ref[1] + hp_ref[...]
    o = jnp.maximum(a * di + b_ref[...], 0.0)
    hp2_ref[...] = jnp.dot(o, w2_ref[...], preferred_element_type=jnp.float32) * di


def _tc_mid(accp, hp, dinv, b, W2):
    return pl.pallas_call(
        _mid_body,
        grid=(GRID,),
        in_specs=[
            pl.BlockSpec((2, BR, D), lambda i: (0, i, 0)),
            pl.BlockSpec((BR, D), lambda i: (i, 0)),
            pl.BlockSpec((BR, 1), lambda i: (i, 0)),
            pl.BlockSpec((1, D), lambda i: (0, 0)),
            pl.BlockSpec((D, D), lambda i: (0, 0)),
        ],
        out_specs=pl.BlockSpec((BR, D), lambda i: (i, 0)),
        out_shape=jax.ShapeDtypeStruct((NP, D), jnp.float32),
    )(accp, hp, dinv, b, W2)


def _head_body(acc_ref, hp_ref, dinv_ref, b_ref, wo_ref, bo_ref, y_ref):
    a = acc_ref[0] + acc_ref[1] + hp_ref[...]
    o = jnp.maximum(a * dinv_ref[...] + b_ref[...], 0.0)
    y_ref[...] = jnp.dot(o, wo_ref[...], preferred_element_type=jnp.float32) + bo_ref[...]


def _tc_head(accp, hp, dinv, b, Wo, bo):
    return pl.pallas_call(
        _head_body,
        grid=(GRID,),
        in_specs=[
            pl.BlockSpec((2, BR, D), lambda i: (0, i, 0)),
            pl.BlockSpec((BR, D), lambda i: (i, 0)),
            pl.BlockSpec((BR, 1), lambda i: (i, 0)),
            pl.BlockSpec((1, D), lambda i: (0, 0)),
            pl.BlockSpec((D, 1), lambda i: (0, 0)),
            pl.BlockSpec((1, 1), lambda i: (0, 0)),
        ],
        out_specs=pl.BlockSpec((BR, 1), lambda i: (i, 0)),
        out_shape=jax.ShapeDtypeStruct((NP, 1), jnp.float32),
    )(accp, hp, dinv, b, Wo, bo)


def kernel(x, edge_index, W1, b1, W2, b2, Wo, bo):
    x_p = jnp.pad(x, ((0, NP - N), (0, 0)))
    pad = jnp.full((EP - E,), NP - 1, jnp.int32)
    src_p = jnp.concatenate([edge_index[0], pad])
    dst_p = jnp.concatenate([edge_index[1], pad])

    degp = _deg_kernel(dst_p).reshape(NW, NP)
    hp1, dinv = _tc_scale1(x_p, W1, degp)
    acc1 = _agg_kernel(hp1, src_p, dst_p).reshape(NC, NP, D)
    hp2 = _tc_mid(acc1, hp1, dinv, b1.reshape(1, D), W2)
    acc2 = _agg_kernel(hp2, src_p, dst_p).reshape(NC, NP, D)
    y = _tc_head(acc2, hp2, dinv, b2.reshape(1, D), Wo, bo.reshape(1, 1))
    return y[:N, 0]
